# baseline (speedup 1.0000x reference)
import functools

import jax
import jax.numpy as jnp
from jax import lax
from jax.experimental import pallas as pl
from jax.experimental.pallas import tpu as pltpu
from jax.experimental.pallas import tpu_sc as plsc

N = 10000
D = 128
NC = 2
NS = 16
L = 16
NW = NC * NS
CH = 128
N_PAD = 10112
ROWS_PER_TILE = N_PAD // NS
TBL_STRIPE = 632
NHALF = 5


def _sc_gather(idx2, pq, chunk0, n_my_chunks):
    k_per_t = n_my_chunks // NS
    npass = 2
    k_pass = k_per_t // npass
    mesh = plsc.VectorSubcoreMesh(core_axis_name="c", subcore_axis_name="s")

    @functools.partial(
        pl.kernel,
        out_type=jax.ShapeDtypeStruct((NC, n_my_chunks * CH, D), jnp.float32),
        mesh=mesh,
        scratch_types=[
            pltpu.VMEM((k_pass, CH), jnp.int32),
            pltpu.VMEM((CH, D), jnp.float32),
            pltpu.VMEM((CH, D), jnp.float32),
            pltpu.VMEM_SHARED((N, D), jnp.float32),
            pltpu.SemaphoreType.DMA,
            pltpu.SemaphoreType.DMA,
            pltpu.SemaphoreType.DMA,
            pltpu.SemaphoreType.DMA,
        ],
    )
    def k(idx_hbm, pq_hbm, out_hbm, idx_v, r0, r1, tbl,
          sg0, sg1, sw0, sw1):
        c = lax.axis_index("c")
        s = lax.axis_index("s")

        @pl.when(s < NS - 1)
        def _():
            pltpu.sync_copy(pq_hbm.at[c, pl.ds(s * TBL_STRIPE, TBL_STRIPE)],
                            tbl.at[pl.ds(s * TBL_STRIPE, TBL_STRIPE)])

        @pl.when(s == NS - 1)
        def _():
            last = N - (NS - 1) * TBL_STRIPE
            pltpu.sync_copy(
                pq_hbm.at[c, pl.ds((NS - 1) * TBL_STRIPE, last)],
                tbl.at[pl.ds((NS - 1) * TBL_STRIPE, last)])

        plsc.subcore_barrier()

        rs, sgs, sws = (r0, r1), (sg0, sg1), (sw0, sw1)

        def phase(obase, j, slot):
            @pl.when(j >= 2)
            def _():
                pltpu.make_async_copy(
                    rs[slot], out_hbm.at[c, pl.ds(obase * CH, CH)],
                    sws[slot]).wait()

            pltpu.async_copy(tbl.at[idx_v.at[j]], rs[slot], sgs[slot]).wait()
            pltpu.async_copy(rs[slot],
                             out_hbm.at[c, pl.ds((obase + j) * CH, CH)],
                             sws[slot])

        for p in range(npass):
            obase = s * k_per_t + p * k_pass
            pltpu.sync_copy(idx_hbm.at[c, pl.ds(chunk0 + obase, k_pass)],
                            idx_v)

            def body(jj, carry):
                phase(obase, 2 * jj, 0)
                phase(obase, 2 * jj + 1, 1)
                return carry

            lax.fori_loop(0, k_pass // 2, body, 0)
            for t in range(2):
                pltpu.make_async_copy(
                    rs[t], out_hbm.at[c, pl.ds(obase * CH, CH)],
                    sws[t]).wait()

    return k(idx2, pq)


def _sc_scatter(dst2d, m, chunk0, n_my_chunks):
    k_per_w = n_my_chunks // NW
    chunks_per_core = n_my_chunks // NC
    mesh = plsc.VectorSubcoreMesh(core_axis_name="c", subcore_axis_name="s")

    @functools.partial(
        pl.kernel,
        out_type=jax.ShapeDtypeStruct((NC, N_PAD, D), jnp.float32),
        mesh=mesh,
        scratch_types=[
            pltpu.VMEM((k_per_w, CH), jnp.int32),
            pltpu.VMEM((CH, D), jnp.float32),
            pltpu.VMEM((CH, D), jnp.float32),
            pltpu.VMEM_SHARED((N_PAD, D), jnp.float32),
            pltpu.SemaphoreType.DMA,
            pltpu.SemaphoreType.DMA,
        ],
    )
    def k(dst_hbm, m_hbm, out_hbm, idx_v, mb0, mb1, acc, srd0, srd1):
        c = lax.axis_index("c")
        s = lax.axis_index("s")

        @plsc.parallel_loop(0, CH)
        def zrow(r):
            for col in range(D // L):
                mb0[r, pl.ds(col * L, L)] = jnp.zeros((L,), jnp.float32)

        zb = s * ROWS_PER_TILE
        for t in range(ROWS_PER_TILE // CH):
            pltpu.sync_copy(mb0, acc.at[pl.ds(zb + t * CH, CH)])
        rem = ROWS_PER_TILE % CH
        if rem:
            pltpu.sync_copy(mb0.at[pl.ds(0, rem)],
                            acc.at[pl.ds(zb + (ROWS_PER_TILE // CH) * CH, rem)])
        plsc.subcore_barrier()

        wk = c * chunks_per_core + s * k_per_w
        pltpu.sync_copy(dst_hbm.at[pl.ds(chunk0 + wk, k_per_w)], idx_v)

        mbs, srds = (mb0, mb1), (srd0, srd1)

        def issue_rd(j, slot):
            pltpu.async_copy(m_hbm.at[pl.ds((wk + j) * CH, CH)], mbs[slot],
                             srds[slot])

        issue_rd(0, 0)
        issue_rd(1, 1)

        def phase(j, slot):
            pltpu.make_async_copy(m_hbm.at[pl.ds((wk + j) * CH, CH)],
                                  mbs[slot], srds[slot]).wait()
            pltpu.sync_copy(mbs[slot], acc.at[idx_v.at[j]], add=True)

            @pl.when(j + 2 < k_per_w)
            def _():
                issue_rd(j + 2, slot)

        def body(jj, carry):
            phase(2 * jj, 0)
            phase(2 * jj + 1, 1)
            return carry

        lax.fori_loop(0, k_per_w // 2, body, 0)
        plsc.subcore_barrier()
        pltpu.sync_copy(acc.at[pl.ds(zb, ROWS_PER_TILE)],
                        out_hbm.at[c, pl.ds(zb, ROWS_PER_TILE)])

    return k(dst2d, m)


def _tc_head(x, w_in, b_in, w1a, w1b, b1):
    def body(x_r, wi_r, bi_r, wa_r, wb_r, b1_r, pq_r):
        h = jnp.dot(x_r[...], wi_r[...], preferred_element_type=jnp.float32) + bi_r[...]
        pq_r[0] = jnp.dot(h, wa_r[...], preferred_element_type=jnp.float32) + b1_r[...]
        pq_r[1] = jnp.dot(h, wb_r[...], preferred_element_type=jnp.float32)

    return pl.pallas_call(
        body,
        out_shape=jax.ShapeDtypeStruct((NC, N, D), jnp.float32),
    )(x, w_in, b_in.reshape(1, D), w1a, w1b, b1.reshape(1, D))


def _sum_parts(part_refs):
    h = part_refs[0][0] + part_refs[0][1]
    for pr in part_refs[1:]:
        h = h + pr[0] + pr[1]
    return h


_NBLK = 2000


def _tc_mid(parts_list, w1a, w1b, b1):
    k = len(parts_list)

    def body(*refs):
        wa_r, wb_r, b1_r, pq_r = refs[k:]
        h = _sum_parts(refs[:k])
        pq_r[0] = jnp.dot(h, wa_r[...], preferred_element_type=jnp.float32) + b1_r[...]
        pq_r[1] = jnp.dot(h, wb_r[...], preferred_element_type=jnp.float32)

    return pl.pallas_call(
        body,
        grid=(N // _NBLK,),
        in_specs=[pl.BlockSpec((NC, _NBLK, D), lambda i: (0, i, 0))] * k + [
            pl.BlockSpec((D, D), lambda i: (0, 0)),
            pl.BlockSpec((D, D), lambda i: (0, 0)),
            pl.BlockSpec((1, D), lambda i: (0, 0)),
        ],
        out_specs=pl.BlockSpec((NC, _NBLK, D), lambda i: (0, i, 0)),
        out_shape=jax.ShapeDtypeStruct((NC, N, D), jnp.float32),
    )(*parts_list, w1a, w1b, b1.reshape(1, D))


def _tc_tail(parts_list, w_out, b_out):
    k = len(parts_list)

    def body(*refs):
        w_r, b1_r, o_r = refs[k:]
        h = _sum_parts(refs[:k])
        o_r[...] = jnp.dot(h, w_r[...], preferred_element_type=jnp.float32) + b1_r[...]

    return pl.pallas_call(
        body,
        grid=(N // _NBLK,),
        in_specs=[pl.BlockSpec((NC, _NBLK, D), lambda i: (0, i, 0))] * k + [
            pl.BlockSpec((D, D), lambda i: (0, 0)),
            pl.BlockSpec((1, D), lambda i: (0, 0)),
        ],
        out_specs=pl.BlockSpec((_NBLK, D), lambda i: (i, 0)),
        out_shape=jax.ShapeDtypeStruct((N, D), jnp.float32),
    )(*parts_list, w_out, b_out.reshape(1, D))


def _tc_edge_mlp(gq, w2, b2, w3, b3, n_rows):
    blk = 2048
    grid = n_rows // blk

    def body(pg_r, qg_r, w2_r, b2_r, w3_r, b3_r, m_r):
        h = jax.nn.relu(pg_r[0] + qg_r[0])
        h = jax.nn.relu(jnp.dot(h, w2_r[...], preferred_element_type=jnp.float32)
                        + b2_r[...])
        m_r[...] = jnp.dot(h, w3_r[...], preferred_element_type=jnp.float32) + b3_r[...]

    return pl.pallas_call(
        body,
        grid=(grid,),
        in_specs=[
            pl.BlockSpec((1, blk, D), lambda i: (0, i, 0)),
            pl.BlockSpec((1, blk, D), lambda i: (1, i, 0)),
            pl.BlockSpec((D, D), lambda i: (0, 0)),
            pl.BlockSpec((1, D), lambda i: (0, 0)),
            pl.BlockSpec((D, D), lambda i: (0, 0)),
            pl.BlockSpec((1, D), lambda i: (0, 0)),
        ],
        out_specs=pl.BlockSpec((blk, D), lambda i: (i, 0)),
        out_shape=jax.ShapeDtypeStruct((n_rows, D), jnp.float32),
    )(gq, gq, w2, b2.reshape(1, D), w3, b3.reshape(1, D))


def kernel(x, lframes, edge_index, w_in, b_in,
           blk0_w1, blk0_b1, blk0_w2, blk0_b2, blk0_w3, blk0_b3,
           blk1_w1, blk1_b1, blk1_w2, blk1_b2, blk1_w3, blk1_b3,
           w_out, b_out):
    e = edge_index.shape[1]
    quantum = NHALF * NW * CH * 8
    e_pad = ((e + quantum - 1) // quantum) * quantum
    pad = e_pad - e
    n_chunks = e_pad // CH
    half_chunks = n_chunks // NHALF

    src = edge_index[0]
    dst = edge_index[1]
    zpad = jnp.zeros((pad,), jnp.int32)
    idx2 = jnp.stack([
        jnp.concatenate([src, zpad]).reshape(n_chunks, CH),
        jnp.concatenate([dst, zpad]).reshape(n_chunks, CH),
    ])
    dsts2d = jnp.concatenate([dst, jnp.full((pad,), N, jnp.int32)]).reshape(
        n_chunks, CH)

    w1a0, w1b0 = blk0_w1[:D], blk0_w1[D:]
    w1a1, w1b1 = blk1_w1[:D], blk1_w1[D:]

    def run_block(pq, w2, b2, w3, b3):
        parts = []
        for hf in range(NHALF):
            gq = _sc_gather(idx2, pq, hf * half_chunks, half_chunks)
            m = _tc_edge_mlp(gq, w2, b2, w3, b3, half_chunks * CH)
            parts.append(_sc_scatter(dsts2d, m, hf * half_chunks, half_chunks))
        return parts

    pq = _tc_head(x, w_in, b_in, w1a0, w1b0, blk0_b1)
    parts = run_block(pq, blk0_w2, blk0_b2, blk0_w3, blk0_b3)
    pq = _tc_mid([p[:, :N] for p in parts], w1a1, w1b1, blk1_b1)
    parts = run_block(pq, blk1_w2, blk1_b2, blk1_w3, blk1_b3)
    return _tc_tail([p[:, :N] for p in parts], w_out, b_out)


# speedup vs baseline: 1.1476x; 1.1476x over previous
import functools

import jax
import jax.numpy as jnp
from jax import lax
from jax.experimental import pallas as pl
from jax.experimental.pallas import tpu as pltpu
from jax.experimental.pallas import tpu_sc as plsc

N = 10000
D = 128
NC = 2
NS = 16
L = 16
NW = NC * NS
CH = 128
N_PAD = 10112
ROWS_PER_TILE = N_PAD // NS
TBL_STRIPE = 632
NHALF = 2


def _sc_gather(idx2, pq, chunk0, n_my_chunks):
    k_per_t = n_my_chunks // NS
    npass = 2
    k_pass = k_per_t // npass
    mesh = plsc.VectorSubcoreMesh(core_axis_name="c", subcore_axis_name="s")

    @functools.partial(
        pl.kernel,
        out_type=jax.ShapeDtypeStruct((NC, n_my_chunks * CH, D), jnp.float32),
        mesh=mesh,
        scratch_types=[
            pltpu.VMEM((k_pass, CH), jnp.int32),
            pltpu.VMEM((CH, D), jnp.float32),
            pltpu.VMEM((CH, D), jnp.float32),
            pltpu.VMEM_SHARED((N, D), jnp.float32),
            pltpu.SemaphoreType.DMA,
            pltpu.SemaphoreType.DMA,
            pltpu.SemaphoreType.DMA,
            pltpu.SemaphoreType.DMA,
        ],
    )
    def k(idx_hbm, pq_hbm, out_hbm, idx_v, r0, r1, tbl,
          sg0, sg1, sw0, sw1):
        c = lax.axis_index("c")
        s = lax.axis_index("s")

        @pl.when(s < NS - 1)
        def _():
            pltpu.sync_copy(pq_hbm.at[c, pl.ds(s * TBL_STRIPE, TBL_STRIPE)],
                            tbl.at[pl.ds(s * TBL_STRIPE, TBL_STRIPE)])

        @pl.when(s == NS - 1)
        def _():
            last = N - (NS - 1) * TBL_STRIPE
            pltpu.sync_copy(
                pq_hbm.at[c, pl.ds((NS - 1) * TBL_STRIPE, last)],
                tbl.at[pl.ds((NS - 1) * TBL_STRIPE, last)])

        plsc.subcore_barrier()

        rs, sgs, sws = (r0, r1), (sg0, sg1), (sw0, sw1)

        def phase(obase, j, slot):
            @pl.when(j >= 2)
            def _():
                pltpu.make_async_copy(
                    rs[slot], out_hbm.at[c, pl.ds(obase * CH, CH)],
                    sws[slot]).wait()

            pltpu.async_copy(tbl.at[idx_v.at[j]], rs[slot], sgs[slot]).wait()
            pltpu.async_copy(rs[slot],
                             out_hbm.at[c, pl.ds((obase + j) * CH, CH)],
                             sws[slot])

        for p in range(npass):
            obase = s * k_per_t + p * k_pass
            pltpu.sync_copy(idx_hbm.at[c, pl.ds(chunk0 + obase, k_pass)],
                            idx_v)

            def body(jj, carry):
                phase(obase, 2 * jj, 0)
                phase(obase, 2 * jj + 1, 1)
                return carry

            lax.fori_loop(0, k_pass // 2, body, 0)
            for t in range(2):
                pltpu.make_async_copy(
                    rs[t], out_hbm.at[c, pl.ds(obase * CH, CH)],
                    sws[t]).wait()

    return k(idx2, pq)


def _sc_scatter(dst2d, m, chunk0, n_my_chunks):
    k_per_w = n_my_chunks // NW
    chunks_per_core = n_my_chunks // NC
    mesh = plsc.VectorSubcoreMesh(core_axis_name="c", subcore_axis_name="s")

    @functools.partial(
        pl.kernel,
        out_type=jax.ShapeDtypeStruct((NC, N_PAD, D), jnp.float32),
        mesh=mesh,
        scratch_types=[
            pltpu.VMEM((k_per_w, CH), jnp.int32),
            pltpu.VMEM((CH, D), jnp.float32),
            pltpu.VMEM((CH, D), jnp.float32),
            pltpu.VMEM_SHARED((N_PAD, D), jnp.float32),
            pltpu.SemaphoreType.DMA,
            pltpu.SemaphoreType.DMA,
        ],
    )
    def k(dst_hbm, m_hbm, out_hbm, idx_v, mb0, mb1, acc, srd0, srd1):
        c = lax.axis_index("c")
        s = lax.axis_index("s")

        @plsc.parallel_loop(0, CH)
        def zrow(r):
            for col in range(D // L):
                mb0[r, pl.ds(col * L, L)] = jnp.zeros((L,), jnp.float32)

        zb = s * ROWS_PER_TILE
        for t in range(ROWS_PER_TILE // CH):
            pltpu.sync_copy(mb0, acc.at[pl.ds(zb + t * CH, CH)])
        rem = ROWS_PER_TILE % CH
        if rem:
            pltpu.sync_copy(mb0.at[pl.ds(0, rem)],
                            acc.at[pl.ds(zb + (ROWS_PER_TILE // CH) * CH, rem)])
        plsc.subcore_barrier()

        wk = c * chunks_per_core + s * k_per_w
        pltpu.sync_copy(dst_hbm.at[pl.ds(chunk0 + wk, k_per_w)], idx_v)

        mbs, srds = (mb0, mb1), (srd0, srd1)

        def issue_rd(j, slot):
            pltpu.async_copy(m_hbm.at[pl.ds((wk + j) * CH, CH)], mbs[slot],
                             srds[slot])

        issue_rd(0, 0)
        issue_rd(1, 1)

        def phase(j, slot):
            pltpu.make_async_copy(m_hbm.at[pl.ds((wk + j) * CH, CH)],
                                  mbs[slot], srds[slot]).wait()
            pltpu.sync_copy(mbs[slot], acc.at[idx_v.at[j]], add=True)

            @pl.when(j + 2 < k_per_w)
            def _():
                issue_rd(j + 2, slot)

        def body(jj, carry):
            phase(2 * jj, 0)
            phase(2 * jj + 1, 1)
            return carry

        lax.fori_loop(0, k_per_w // 2, body, 0)
        plsc.subcore_barrier()
        pltpu.sync_copy(acc.at[pl.ds(zb, ROWS_PER_TILE)],
                        out_hbm.at[c, pl.ds(zb, ROWS_PER_TILE)])

    return k(dst2d, m)


def _tc_head(x, w_in, b_in, w1a, w1b, b1):
    def body(x_r, wi_r, bi_r, wa_r, wb_r, b1_r, pq_r):
        h = jnp.dot(x_r[...], wi_r[...], preferred_element_type=jnp.float32) + bi_r[...]
        pq_r[0] = jnp.dot(h, wa_r[...], preferred_element_type=jnp.float32) + b1_r[...]
        pq_r[1] = jnp.dot(h, wb_r[...], preferred_element_type=jnp.float32)

    return pl.pallas_call(
        body,
        out_shape=jax.ShapeDtypeStruct((NC, N, D), jnp.float32),
    )(x, w_in, b_in.reshape(1, D), w1a, w1b, b1.reshape(1, D))


def _sum_parts(part_refs):
    h = part_refs[0][0] + part_refs[0][1]
    for pr in part_refs[1:]:
        h = h + pr[0] + pr[1]
    return h


_NBLK = 2000


def _tc_mid(parts_list, w1a, w1b, b1):
    k = len(parts_list)

    def body(*refs):
        wa_r, wb_r, b1_r, pq_r = refs[k:]
        h = _sum_parts(refs[:k])
        pq_r[0] = jnp.dot(h, wa_r[...], preferred_element_type=jnp.float32) + b1_r[...]
        pq_r[1] = jnp.dot(h, wb_r[...], preferred_element_type=jnp.float32)

    return pl.pallas_call(
        body,
        grid=(N // _NBLK,),
        in_specs=[pl.BlockSpec((NC, _NBLK, D), lambda i: (0, i, 0))] * k + [
            pl.BlockSpec((D, D), lambda i: (0, 0)),
            pl.BlockSpec((D, D), lambda i: (0, 0)),
            pl.BlockSpec((1, D), lambda i: (0, 0)),
        ],
        out_specs=pl.BlockSpec((NC, _NBLK, D), lambda i: (0, i, 0)),
        out_shape=jax.ShapeDtypeStruct((NC, N, D), jnp.float32),
    )(*parts_list, w1a, w1b, b1.reshape(1, D))


def _tc_tail(parts_list, w_out, b_out):
    k = len(parts_list)

    def body(*refs):
        w_r, b1_r, o_r = refs[k:]
        h = _sum_parts(refs[:k])
        o_r[...] = jnp.dot(h, w_r[...], preferred_element_type=jnp.float32) + b1_r[...]

    return pl.pallas_call(
        body,
        grid=(N // _NBLK,),
        in_specs=[pl.BlockSpec((NC, _NBLK, D), lambda i: (0, i, 0))] * k + [
            pl.BlockSpec((D, D), lambda i: (0, 0)),
            pl.BlockSpec((1, D), lambda i: (0, 0)),
        ],
        out_specs=pl.BlockSpec((_NBLK, D), lambda i: (i, 0)),
        out_shape=jax.ShapeDtypeStruct((N, D), jnp.float32),
    )(*parts_list, w_out, b_out.reshape(1, D))


def _tc_edge_mlp(gq, w2, b2, w3, b3, n_rows):
    blk = 2048
    grid = n_rows // blk

    def body(pg_r, qg_r, w2_r, b2_r, w3_r, b3_r, m_r):
        h = jax.nn.relu(pg_r[0] + qg_r[0])
        h = jax.nn.relu(jnp.dot(h, w2_r[...], preferred_element_type=jnp.float32)
                        + b2_r[...])
        m_r[...] = jnp.dot(h, w3_r[...], preferred_element_type=jnp.float32) + b3_r[...]

    return pl.pallas_call(
        body,
        grid=(grid,),
        in_specs=[
            pl.BlockSpec((1, blk, D), lambda i: (0, i, 0)),
            pl.BlockSpec((1, blk, D), lambda i: (1, i, 0)),
            pl.BlockSpec((D, D), lambda i: (0, 0)),
            pl.BlockSpec((1, D), lambda i: (0, 0)),
            pl.BlockSpec((D, D), lambda i: (0, 0)),
            pl.BlockSpec((1, D), lambda i: (0, 0)),
        ],
        out_specs=pl.BlockSpec((blk, D), lambda i: (i, 0)),
        out_shape=jax.ShapeDtypeStruct((n_rows, D), jnp.float32),
    )(gq, gq, w2, b2.reshape(1, D), w3, b3.reshape(1, D))


def kernel(x, lframes, edge_index, w_in, b_in,
           blk0_w1, blk0_b1, blk0_w2, blk0_b2, blk0_w3, blk0_b3,
           blk1_w1, blk1_b1, blk1_w2, blk1_b2, blk1_w3, blk1_b3,
           w_out, b_out):
    e = edge_index.shape[1]
    quantum = NHALF * NW * CH * 8
    e_pad = ((e + quantum - 1) // quantum) * quantum
    pad = e_pad - e
    n_chunks = e_pad // CH
    half_chunks = n_chunks // NHALF

    src = edge_index[0]
    dst = edge_index[1]
    zpad = jnp.zeros((pad,), jnp.int32)
    idx2 = jnp.stack([
        jnp.concatenate([src, zpad]).reshape(n_chunks, CH),
        jnp.concatenate([dst, zpad]).reshape(n_chunks, CH),
    ])
    dsts2d = jnp.concatenate([dst, jnp.full((pad,), N, jnp.int32)]).reshape(
        n_chunks, CH)

    w1a0, w1b0 = blk0_w1[:D], blk0_w1[D:]
    w1a1, w1b1 = blk1_w1[:D], blk1_w1[D:]

    def run_block(pq, w2, b2, w3, b3):
        parts = []
        for hf in range(NHALF):
            gq = _sc_gather(idx2, pq, hf * half_chunks, half_chunks)
            m = _tc_edge_mlp(gq, w2, b2, w3, b3, half_chunks * CH)
            parts.append(_sc_scatter(dsts2d, m, hf * half_chunks, half_chunks))
        return parts

    pq = _tc_head(x, w_in, b_in, w1a0, w1b0, blk0_b1)
    parts = run_block(pq, blk0_w2, blk0_b2, blk0_w3, blk0_b3)
    pq = _tc_mid([p[:, :N] for p in parts], w1a1, w1b1, blk1_b1)
    parts = run_block(pq, blk1_w2, blk1_b2, blk1_w3, blk1_b3)
    return _tc_tail([p[:, :N] for p in parts], w_out, b_out)
